# baseline (speedup 1.0000x reference)
import functools

import jax
import jax.numpy as jnp
import numpy as np
from jax import lax
from jax.experimental import pallas as pl
from jax.experimental.pallas import tpu as pltpu
from jax.experimental.pallas import tpu_sc as plsc

_NC = 19
_EPS = float(np.finfo(np.float32).eps)

_H = 512
_W = 512
_NB = 4
_TC_ROWS = 384
_TC_BLK = 128
_NW = 32
_GRP = 8


def _tc_body(x_ref, t_ref, hist_ref, acc_ref, *, num_steps, rows):
    step = pl.program_id(0)

    @pl.when(step == 0)
    def _init():
        acc_ref[...] = jnp.zeros_like(acc_ref)

    x = x_ref[0]
    t = t_ref[...]
    cls = lax.broadcasted_iota(jnp.int32, (_NC, rows, _W), 0)
    m = jnp.max(x, axis=0, keepdims=True)
    pred = jnp.min(jnp.where(x == m, cls, _NC), axis=0, keepdims=True)
    npx = rows * _W
    t2 = t.reshape(1, npx)
    p2 = pred.reshape(1, npx)
    cls2 = lax.broadcasted_iota(jnp.int32, (_NC, npx), 0)
    a = (cls2 == t2).astype(jnp.bfloat16)
    b = (cls2 == p2).astype(jnp.bfloat16)
    acc_ref[...] += lax.dot_general(
        a, b, (((1,), (1,)), ((), ())),
        preferred_element_type=jnp.float32)

    @pl.when(step == num_steps - 1)
    def _write():
        hist_ref[...] = acc_ref[...]


def _tc_partial_hist(logits, target):
    steps_per_b = _TC_ROWS // _TC_BLK
    num_steps = _NB * steps_per_b
    return pl.pallas_call(
        functools.partial(_tc_body, num_steps=num_steps, rows=_TC_BLK),
        grid=(num_steps,),
        in_specs=[
            pl.BlockSpec((1, _NC, _TC_BLK, _W),
                         lambda i: (i // steps_per_b, 0, i % steps_per_b, 0)),
            pl.BlockSpec((1, _TC_BLK, _W),
                         lambda i: (i // steps_per_b, i % steps_per_b, 0)),
        ],
        out_specs=pl.BlockSpec((_NC, _NC), lambda i: (0, 0)),
        out_shape=jax.ShapeDtypeStruct((_NC, _NC), jnp.float32),
        scratch_shapes=[pltpu.VMEM((_NC, _NC), jnp.float32)],
    )(logits, target)


def _sc_hist_kernel(logit_hbm, tgt_hbm, out_hbm, buf, tbuf, hist, sem):
    sc_rows = _H - _TC_ROWS
    n_grp = sc_rows // _GRP
    n_tasks = _NB * n_grp
    tpw = n_tasks // _NW
    cid = lax.axis_index("c")
    sid = lax.axis_index("s")
    wid = sid * 2 + cid

    zero16 = jnp.zeros((16,), jnp.float32)
    for r in range(0, _NC * 32, 16):
        hist[pl.ds(r, 16)] = zero16

    ones16 = jnp.ones((16,), jnp.float32)
    for k in range(tpw):
        task = wid * tpw + k
        b = task // n_grp
        g = task % n_grp
        r0 = _TC_ROWS + g * _GRP
        copies = []
        for c in range(_NC):
            copies.append(pltpu.async_copy(
                logit_hbm.at[b, c, pl.ds(r0, _GRP), :],
                buf.at[pl.ds(c * _GRP, _GRP), :], sem))
        tcopy = pltpu.async_copy(
            tgt_hbm.at[b, pl.ds(r0, _GRP), :], tbuf, sem)
        for cp in copies:
            cp.wait()
        tcopy.wait()

        def vec_body(i, _):
            row = i >> 5
            lane = (i & 31) * 16
            m = buf[row, pl.ds(lane, 16)]
            am = jnp.zeros((16,), jnp.int32)
            for c in range(1, _NC):
                v = buf[c * _GRP + row, pl.ds(lane, 16)]
                gt = v > m
                m = jnp.where(gt, v, m)
                am = jnp.where(gt, c, am)
            t = tbuf[row, pl.ds(lane, 16)]
            valid = (t >= 0) & (t < _NC)
            ts = jnp.where(valid, t, 0)
            plsc.addupdate_scatter(hist, [ts * 32 + am], ones16, mask=valid)
            return 0

        lax.fori_loop(0, _GRP * _W // 16, vec_body, 0)

    pltpu.sync_copy(hist, out_hbm.at[pl.ds(wid * _NC * 32, _NC * 32)])


def _sc_partial_hist(logits, target):
    mesh = plsc.VectorSubcoreMesh(core_axis_name="c", subcore_axis_name="s")
    f = functools.partial(
        pl.kernel,
        mesh=mesh,
        out_type=jax.ShapeDtypeStruct((_NW * _NC * 32,), jnp.float32),
        scratch_types=[
            pltpu.VMEM((_NC * _GRP, _W), jnp.float32),
            pltpu.VMEM((_GRP, _W), jnp.int32),
            pltpu.VMEM((_NC * 32,), jnp.float32),
            pltpu.SemaphoreType.DMA,
        ],
        compiler_params=pltpu.CompilerParams(needs_layout_passes=False),
    )(_sc_hist_kernel)
    return f(logits, target)


def _final_body(tc_ref, sc_ref, o_ref):
    hist = tc_ref[...] + jnp.sum(sc_ref[...], axis=0)[:, :_NC]
    r0 = lax.broadcasted_iota(jnp.int32, (_NC, _NC), 0)
    r1 = lax.broadcasted_iota(jnp.int32, (_NC, _NC), 1)
    diag = (r0 == r1).astype(jnp.float32)
    tp = jnp.sum(hist * diag, axis=1)
    sum1 = jnp.sum(hist, axis=1)
    sum0 = jnp.sum(hist, axis=0)
    iou = tp / (sum1 + sum0 - tp + _EPS)
    o_ref[...] = jnp.reshape(jnp.sum(iou) * (100.0 / _NC), (1, 1))


def _finalize(hist_tc, hist_sc):
    return pl.pallas_call(
        _final_body,
        out_shape=jax.ShapeDtypeStruct((1, 1), jnp.float32),
    )(hist_tc, hist_sc)


def kernel(input_img, input, target):
    del input_img
    hist_tc = _tc_partial_hist(input, target)
    hist_sc = _sc_partial_hist(input, target)
    return _finalize(hist_tc, hist_sc.reshape(_NW, _NC, 32)).reshape(1)


# speedup vs baseline: 1.2686x; 1.2686x over previous
import functools

import jax
import jax.numpy as jnp
import numpy as np
from jax import lax
from jax.experimental import pallas as pl
from jax.experimental.pallas import tpu as pltpu

_NC = 19
_EPS = float(np.finfo(np.float32).eps)


def _body(x_ref, t_ref, o_ref, acc_ref, *, num_steps, rows):
    step = pl.program_id(0)

    @pl.when(step == 0)
    def _init():
        acc_ref[...] = jnp.zeros_like(acc_ref)

    x = x_ref[0]
    t = t_ref[...]
    cls = lax.broadcasted_iota(jnp.int32, (_NC, rows, 512), 0)
    m = jnp.max(x, axis=0, keepdims=True)
    pred = jnp.min(jnp.where(x == m, cls, _NC), axis=0, keepdims=True)
    npx = rows * 512
    t2 = t.reshape(1, npx)
    p2 = pred.reshape(1, npx)
    cls2 = lax.broadcasted_iota(jnp.int32, (_NC, npx), 0)
    a = (cls2 == t2).astype(jnp.bfloat16)
    b = (cls2 == p2).astype(jnp.bfloat16)
    acc_ref[...] += lax.dot_general(
        a, b, (((1,), (1,)), ((), ())),
        preferred_element_type=jnp.float32)

    @pl.when(step == num_steps - 1)
    def _finalize():
        hist = acc_ref[...]
        r0 = lax.broadcasted_iota(jnp.int32, (_NC, _NC), 0)
        r1 = lax.broadcasted_iota(jnp.int32, (_NC, _NC), 1)
        diag = (r0 == r1).astype(jnp.float32)
        tp = jnp.sum(hist * diag, axis=1)
        sum1 = jnp.sum(hist, axis=1)
        sum0 = jnp.sum(hist, axis=0)
        iou = tp / (sum1 + sum0 - tp + _EPS)
        o_ref[...] = jnp.reshape(jnp.sum(iou) * (100.0 / _NC), (1, 1))


def kernel(input_img, input, target):
    del input_img
    n_b, n_c, h, w = input.shape
    rows = 512
    steps_per_b = h // rows
    num_steps = n_b * steps_per_b

    out = pl.pallas_call(
        functools.partial(_body, num_steps=num_steps, rows=rows),
        grid=(num_steps,),
        in_specs=[
            pl.BlockSpec((1, n_c, rows, w),
                         lambda i: (i // steps_per_b, 0, i % steps_per_b, 0)),
            pl.BlockSpec((1, rows, w),
                         lambda i: (i // steps_per_b, i % steps_per_b, 0)),
        ],
        out_specs=pl.BlockSpec((1, 1), lambda i: (0, 0)),
        out_shape=jax.ShapeDtypeStruct((1, 1), jnp.float32),
        scratch_shapes=[pltpu.VMEM((_NC, _NC), jnp.float32)],
    )(input, target)
    return out.reshape(1)


# speedup vs baseline: 1.3962x; 1.1006x over previous
import functools

import jax
import jax.numpy as jnp
import numpy as np
from jax import lax
from jax.experimental import pallas as pl
from jax.experimental.pallas import tpu as pltpu

_NC = 19
_EPS = float(np.finfo(np.float32).eps)


def _body(x_ref, t_ref, o_ref, acc_ref, *, num_steps, rows):
    step = pl.program_id(0)

    @pl.when(step == 0)
    def _init():
        acc_ref[...] = jnp.zeros_like(acc_ref)

    x = x_ref[0]
    t = t_ref[...]
    cls = lax.broadcasted_iota(jnp.int32, (_NC, rows, 512), 0)
    m = jnp.max(x, axis=0, keepdims=True)
    pred = jnp.min(jnp.where(x == m, cls, _NC), axis=0, keepdims=True)
    npx = rows * 512
    comb = (t * 32 + pred).reshape(1, npx)
    t2 = comb >> 5
    p2 = comb & 31
    cls2 = lax.broadcasted_iota(jnp.int32, (_NC, npx), 0)
    a = (cls2 == t2).astype(jnp.bfloat16)
    b = (cls2 == p2).astype(jnp.bfloat16)
    acc_ref[...] += lax.dot_general(
        a, b, (((1,), (1,)), ((), ())),
        preferred_element_type=jnp.float32)

    @pl.when(step == num_steps - 1)
    def _finalize():
        hist = acc_ref[...]
        r0 = lax.broadcasted_iota(jnp.int32, (_NC, _NC), 0)
        r1 = lax.broadcasted_iota(jnp.int32, (_NC, _NC), 1)
        diag = (r0 == r1).astype(jnp.float32)
        tp = jnp.sum(hist * diag, axis=1)
        sum1 = jnp.sum(hist, axis=1)
        sum0 = jnp.sum(hist, axis=0)
        iou = tp / (sum1 + sum0 - tp + _EPS)
        o_ref[...] = jnp.reshape(jnp.sum(iou) * (100.0 / _NC), (1, 1))


def kernel(input_img, input, target):
    del input_img
    n_b, n_c, h, w = input.shape
    rows = 256
    steps_per_b = h // rows
    num_steps = n_b * steps_per_b

    out = pl.pallas_call(
        functools.partial(_body, num_steps=num_steps, rows=rows),
        grid=(num_steps,),
        in_specs=[
            pl.BlockSpec((1, n_c, rows, w),
                         lambda i: (i // steps_per_b, 0, i % steps_per_b, 0)),
            pl.BlockSpec((1, rows, w),
                         lambda i: (i // steps_per_b, i % steps_per_b, 0)),
        ],
        out_specs=pl.BlockSpec((1, 1), lambda i: (0, 0)),
        out_shape=jax.ShapeDtypeStruct((1, 1), jnp.float32),
        scratch_shapes=[pltpu.VMEM((_NC, _NC), jnp.float32)],
    )(input, target)
    return out.reshape(1)
